# baseline (speedup 1.0000x reference)
import functools

import jax
import jax.numpy as jnp
from jax import lax
from jax.experimental import pallas as pl
from jax.experimental.pallas import tpu as pltpu
from jax.experimental.pallas import tpu_sc as plsc

CYCLE = 168
L_OUT = 200
CH = 64
NC = 2
NS = 16
NW = NC * NS
CHUNK = 128


def _sc_window_gather(base_idx, data):
    B = base_idx.shape[0]
    b_per_w = B // NW
    rows_w = b_per_w * L_OUT
    n_chunks = rows_w // CHUNK

    mesh = plsc.VectorSubcoreMesh(core_axis_name="c", subcore_axis_name="s")

    @functools.partial(
        pl.kernel,
        out_type=jax.ShapeDtypeStruct((B * L_OUT, CH), jnp.float32),
        mesh=mesh,
        compiler_params=pltpu.CompilerParams(
            needs_layout_passes=False, use_tc_tiling_on_sc=False),
        scratch_types=[
            pltpu.VMEM((b_per_w,), jnp.int32),
            pltpu.VMEM((rows_w,), jnp.int32),
            pltpu.VMEM((CHUNK, CH), jnp.float32),
            pltpu.VMEM((CHUNK, CH), jnp.float32),
            pltpu.SemaphoreType.DMA,
            pltpu.SemaphoreType.DMA,
            pltpu.SemaphoreType.DMA,
        ],
    )
    def k(idx_hbm, data_hbm, out_hbm, idx_v, idx_buf, buf_a, buf_b,
          gsem, ssem_a, ssem_b):
        wid = lax.axis_index("s") * NC + lax.axis_index("c")
        b0 = wid * b_per_w
        pltpu.sync_copy(idx_hbm.at[pl.ds(b0, b_per_w)], idx_v)
        lane = lax.broadcasted_iota(jnp.int32, (16,), 0)
        for g in range(b_per_w // 16):
            vec = idx_v[pl.ds(g * 16, 16)]
            offs0 = (g * 16 + lane) * L_OUT

            def build(step, v, offs0=offs0):
                for s in range(8):
                    l = step * 8 + s
                    plsc.store_scatter(idx_buf, [offs0 + l], v)
                    v = v + 1
                    v = jnp.where(v == CYCLE, 0, v)
                return v

            lax.fori_loop(0, L_OUT // 8, build, vec)

        row0 = wid * rows_w
        bufs = (buf_a, buf_b)
        ssems = (ssem_a, ssem_b)

        def out_slice(c):
            return out_hbm.at[pl.ds(row0 + c * CHUNK, CHUNK)]

        def pair(q, carry):
            for b in (0, 1):
                c = 2 * q + b
                buf, ssem = bufs[b], ssems[b]

                @pl.when(q >= 1)
                def _wait_prev(buf=buf, ssem=ssem, c=c):
                    pltpu.make_async_copy(buf, out_slice(c - 2), ssem).wait()

                isl = idx_buf.at[pl.ds(c * CHUNK, CHUNK)]
                pltpu.async_copy(data_hbm.at[isl], buf, gsem).wait()
                pltpu.async_copy(buf, out_slice(c), ssem)
            return carry

        lax.fori_loop(0, n_chunks // 2, pair, 0)
        for b in (0, 1):
            c = n_chunks - 2 + b
            pltpu.make_async_copy(bufs[b], out_slice(c), ssems[b]).wait()

    return k(base_idx, data)


def kernel(index, length, data):
    B = index.shape[0]
    base_idx = ((index.reshape(B).astype(jnp.int32) + (length - L_OUT))
                % CYCLE).astype(jnp.int32)
    out = _sc_window_gather(base_idx, data)
    return out.reshape(B, L_OUT, CH)


# speedup vs baseline: 1.0678x; 1.0678x over previous
import functools

import jax
import jax.numpy as jnp
from jax import lax
from jax.experimental import pallas as pl
from jax.experimental.pallas import tpu as pltpu
from jax.experimental.pallas import tpu_sc as plsc

CYCLE = 168
L_OUT = 200
CH = 64
NC = 2
NS = 16
NW = NC * NS
LANES = 16


def _sc_transposed_windows(base_idx, data):
    B = base_idx.shape[0]
    b_per_w = B // NW
    n_vregs = b_per_w // LANES

    mesh = plsc.VectorSubcoreMesh(core_axis_name="c", subcore_axis_name="s")

    @functools.partial(
        pl.kernel,
        out_type=jax.ShapeDtypeStruct((L_OUT, CH, B), jnp.float32),
        mesh=mesh,
        compiler_params=pltpu.CompilerParams(needs_layout_passes=False),
        scratch_types=[
            pltpu.VMEM((b_per_w,), jnp.int32),
            pltpu.VMEM((CYCLE, CH), jnp.float32),
            pltpu.VMEM((CH, b_per_w), jnp.float32),
            pltpu.VMEM((CH, b_per_w), jnp.float32),
            pltpu.SemaphoreType.DMA,
            pltpu.SemaphoreType.DMA,
        ],
    )
    def k(idx_hbm, data_hbm, out_hbm, idx_v, table_v, buf_a, buf_b,
          sem_a, sem_b):
        wid = lax.axis_index("s") * NC + lax.axis_index("c")
        b0 = wid * b_per_w
        pltpu.sync_copy(data_hbm, table_v)
        pltpu.sync_copy(idx_hbm.at[pl.ds(b0, b_per_w)], idx_v)

        iv0 = tuple(idx_v[pl.ds(k * LANES, LANES)] for k in range(n_vregs))
        bufs = (buf_a, buf_b)
        sems = (sem_a, sem_b)

        def out_block(l):
            return out_hbm.at[l, :, pl.ds(b0, b_per_w)]

        def step(q, l, iv, buf, sem):
            @pl.when(q >= 1)
            def _wait_prev(l=l):
                pltpu.make_async_copy(buf, out_block(l - 2), sem).wait()

            for c in range(CH):
                cs = jnp.full((LANES,), c, jnp.int32)
                for g in range(n_vregs):
                    val = plsc.load_gather(table_v, [iv[g], cs])
                    buf[c, pl.ds(g * LANES, LANES)] = val
            pltpu.async_copy(buf, out_block(l), sem)

            @pl.when(l < L_OUT - CYCLE)
            def _dup(l=l):
                pltpu.async_copy(buf, out_block(l + CYCLE), sem).wait()

            nxt = []
            for g in range(n_vregs):
                v = iv[g] + 1
                nxt.append(jnp.where(v == CYCLE, 0, v))
            return tuple(nxt)

        def pair(q, iv):
            for p in (0, 1):
                iv = step(q, 2 * q + p, iv, bufs[p], sems[p])
            return iv

        lax.fori_loop(0, CYCLE // 2, pair, iv0)
        for p in (0, 1):
            pltpu.make_async_copy(
                bufs[p], out_block(CYCLE - 2 + p), sems[p]).wait()

    return k(base_idx, data)


def kernel(index, length, data):
    B = index.shape[0]
    base_idx = ((index.reshape(B).astype(jnp.int32) + (length - L_OUT))
                % CYCLE).astype(jnp.int32)
    out_t = _sc_transposed_windows(base_idx, data)
    return jnp.transpose(out_t, (2, 0, 1))


# speedup vs baseline: 4.2201x; 3.9522x over previous
import functools

import jax
import jax.numpy as jnp
from jax import lax
from jax.experimental import pallas as pl
from jax.experimental.pallas import tpu as pltpu
from jax.experimental.pallas import tpu_sc as plsc

CYCLE = 168
L_OUT = 200
CH = 64
NC = 2
NS = 16
NW = NC * NS
LANES = 16


def _sc_transposed_windows(base_idx, data):
    B = base_idx.shape[0]
    b_per_w = B // NW
    n_vregs = b_per_w // LANES

    mesh = plsc.VectorSubcoreMesh(core_axis_name="c", subcore_axis_name="s")

    @functools.partial(
        pl.kernel,
        out_type=jax.ShapeDtypeStruct((L_OUT, CH, B), jnp.float32),
        mesh=mesh,
        compiler_params=pltpu.CompilerParams(needs_layout_passes=False),
        scratch_types=[
            pltpu.VMEM((b_per_w,), jnp.int32),
            pltpu.VMEM((CH, CYCLE), jnp.float32),
            pltpu.VMEM((CH, b_per_w), jnp.float32),
            pltpu.VMEM((CH, b_per_w), jnp.float32),
            pltpu.SemaphoreType.DMA,
            pltpu.SemaphoreType.DMA,
        ],
    )
    def k(idx_hbm, data_hbm, out_hbm, idx_v, table_v, buf_a, buf_b,
          sem_a, sem_b):
        wid = lax.axis_index("s") * NC + lax.axis_index("c")
        b0 = wid * b_per_w
        pltpu.sync_copy(data_hbm, table_v)
        pltpu.sync_copy(idx_hbm.at[pl.ds(b0, b_per_w)], idx_v)

        iv0 = tuple(idx_v[pl.ds(k * LANES, LANES)] for k in range(n_vregs))
        bufs = (buf_a, buf_b)
        sems = (sem_a, sem_b)

        def out_block(l):
            return out_hbm.at[l, :, pl.ds(b0, b_per_w)]

        def step(q, l, iv, buf, sem):
            @pl.when(q >= 1)
            def _wait_prev(l=l):
                pltpu.make_async_copy(buf, out_block(l - 2), sem).wait()

            for c0 in range(0, CH, 2):
                vals = []
                for cc in (0, 1):
                    cs = jnp.full((LANES,), c0 + cc, jnp.int32)
                    for g in range(n_vregs):
                        vals.append(plsc.load_gather(table_v, [cs, iv[g]]))
                i = 0
                for cc in (0, 1):
                    for g in range(n_vregs):
                        buf[c0 + cc, pl.ds(g * LANES, LANES)] = vals[i]
                        i += 1
            pltpu.async_copy(buf, out_block(l), sem)

            @pl.when(l < L_OUT - CYCLE)
            def _dup(l=l):
                pltpu.async_copy(buf, out_block(l + CYCLE), sem).wait()

            nxt = []
            for g in range(n_vregs):
                v = iv[g] + 1
                nxt.append(jnp.where(v == CYCLE, 0, v))
            return tuple(nxt)

        def pair(q, iv):
            for p in (0, 1):
                iv = step(q, 2 * q + p, iv, bufs[p], sems[p])
            return iv

        lax.fori_loop(0, CYCLE // 2, pair, iv0)
        for p in (0, 1):
            pltpu.make_async_copy(
                bufs[p], out_block(CYCLE - 2 + p), sems[p]).wait()

    return k(base_idx, data)


def kernel(index, length, data):
    B = index.shape[0]
    base_idx = ((index.reshape(B).astype(jnp.int32) + (length - L_OUT))
                % CYCLE).astype(jnp.int32)
    out_t = _sc_transposed_windows(base_idx, data.T)
    return jnp.transpose(out_t, (2, 0, 1))
